# baseline (speedup 1.0000x reference)
import functools

import jax
import jax.numpy as jnp
from jax import lax
from jax.experimental import pallas as pl
from jax.experimental.pallas import tpu as pltpu
from jax.experimental.pallas import tpu_sc as plsc

N_NODES = 10000
D = 128
E = 320000
DELTA = 1.0

SEG = 10240
NC = 2
NS = 16
L = 16
NW = NC * NS
EPW = E // NW
C = 80
NCHUNK = EPW // C
GROUPS = C // L
NEG_INIT = -3.0e38


def _shift(tmp_v, x, idx):
  tmp_v[pl.ds(0, L)] = x
  return plsc.load_gather(tmp_v, [idx])


def _seg_sum_update(keys, dots, ii, sum_v, cnt_v, ktmp_v, vtmp_v, ctmp_v):
  ks, vs = plsc.sort_key_val(keys, dots)
  ktmp_v[pl.ds(0, L)] = ks
  k_prev = plsc.load_gather(ktmp_v, [jnp.maximum(ii - 1, 0)])
  is_head = (ii == 0) | (ks != k_prev)
  head = plsc.cummax(jnp.where(is_head, ii, 0))
  csum = plsc.cumsum(vs)
  vtmp_v[pl.ds(0, L)] = csum
  s_prev = plsc.load_gather(vtmp_v, [jnp.maximum(head - 1, 0)])
  run_sum = csum - jnp.where(head > 0, s_prev, 0.0)
  run_cnt = (ii - head + 1).astype(jnp.float32)
  k_next = plsc.load_gather(ktmp_v, [jnp.minimum(ii + 1, L - 1)])
  is_tail = (ii == L - 1) | (ks != k_next)
  old_s = plsc.load_gather(sum_v, [ks])
  plsc.store_scatter(sum_v, [ks], old_s + run_sum, mask=is_tail)
  old_c = plsc.load_gather(cnt_v, [ks])
  plsc.store_scatter(cnt_v, [ks], old_c + run_cnt, mask=is_tail)


def _seg_max_update(keys, dots, ii, max_v, flag_v, ktmp_v, vtmp_v):
  vs1, ks1 = plsc.sort_key_val(dots, keys, descending=True)
  ks, vs = plsc.sort_key_val(ks1, vs1)
  ktmp_v[pl.ds(0, L)] = ks
  k_prev = plsc.load_gather(ktmp_v, [jnp.maximum(ii - 1, 0)])
  is_head = (ii == 0) | (ks != k_prev)
  old_m = plsc.load_gather(max_v, [ks])
  plsc.store_scatter(max_v, [ks], jnp.maximum(old_m, vs), mask=is_head)
  plsc.store_scatter(flag_v, [ks], jnp.ones((L,), jnp.float32), mask=is_head)


_MESH = plsc.VectorSubcoreMesh(core_axis_name="c", subcore_axis_name="s")


@functools.partial(
    pl.kernel,
    out_type=[jax.ShapeDtypeStruct((NW, SEG), jnp.float32)] * 4,
    mesh=_MESH,
    compiler_params=pltpu.CompilerParams(
        needs_layout_passes=False, use_tc_tiling_on_sc=False),
    scratch_types=[
        pltpu.VMEM((N_NODES,), jnp.int32),
        pltpu.VMEM((SEG,), jnp.float32),
        pltpu.VMEM((SEG,), jnp.float32),
        pltpu.VMEM((SEG,), jnp.float32),
        pltpu.VMEM((SEG,), jnp.float32),
        pltpu.VMEM((2, C), jnp.int32),
        pltpu.VMEM((2, C), jnp.int32),
        pltpu.VMEM((C,), jnp.int32),
        pltpu.VMEM((C,), jnp.int32),
        pltpu.VMEM((C,), jnp.int32),
        pltpu.VMEM((C,), jnp.int32),
        pltpu.VMEM((C,), jnp.int32),
        pltpu.VMEM((C,), jnp.int32),
        pltpu.VMEM((C, D // 2), jnp.int32),
        pltpu.VMEM((C, D // 2), jnp.int32),
        pltpu.VMEM((C, D // 2), jnp.int32),
        pltpu.VMEM((C, D // 2), jnp.int32),
        pltpu.VMEM((L,), jnp.int32),
        pltpu.VMEM((L,), jnp.float32),
        pltpu.VMEM((L,), jnp.float32),
        pltpu.VMEM((L * 17,), jnp.float32),
        pltpu.SemaphoreType.DMA,
        pltpu.SemaphoreType.DMA,
        pltpu.SemaphoreType.DMA,
        pltpu.SemaphoreType.DMA,
        pltpu.SemaphoreType.DMA,
        pltpu.SemaphoreType.DMA,
    ],
)
def _sc_partials(emb_hbm, inv_hbm, pos_hbm, neg_hbm,
                 psum_out, pcnt_out, nmax_out, ncnt_out,
                 inv_v, psum_v, pcnt_v, nmax_v, ncnt_v,
                 edges_p, edges_n, keys_p, keys_n,
                 fsrc_p, fdst_p, fsrc_n, fdst_n,
                 rows_pa, rows_pb, rows_na, rows_nb,
                 ktmp_v, vtmp_v, ctmp_v, tr_v,
                 sem_ep, sem_en, sem_rpa, sem_rpb, sem_rna, sem_rnb):
  wid = lax.axis_index("s") * NC + lax.axis_index("c")
  ii = lax.iota(jnp.int32, L)
  ebase = wid * EPW

  zero16 = jnp.zeros((L,), jnp.float32)
  neg16 = jnp.full((L,), NEG_INIT, jnp.float32)

  def init_body(i, carry):
    off = i * L
    psum_v[pl.ds(off, L)] = zero16
    pcnt_v[pl.ds(off, L)] = zero16
    nmax_v[pl.ds(off, L)] = neg16
    ncnt_v[pl.ds(off, L)] = zero16
    return carry

  lax.fori_loop(0, SEG // L, init_body, 0)

  pltpu.sync_copy(inv_hbm, inv_v)

  def fire_edges(e_hbm, ci, ebuf, sem):
    base = ebase + ci * C
    pltpu.async_copy(e_hbm.at[pl.ds(0, 2), pl.ds(base, C)], ebuf, sem)

  def wait_edges(e_hbm, ebuf, sem):
    pltpu.make_async_copy(e_hbm.at[pl.ds(0, 2), pl.ds(0, C)], ebuf,
                          sem).wait()

  def translate(ebuf, kv, fs, fd):
    for g in range(GROUPS):
      s16 = ebuf[0, pl.ds(g * L, L)]
      d16 = ebuf[1, pl.ds(g * L, L)]
      fs[pl.ds(g * L, L)] = plsc.load_gather(inv_v, [s16])
      fd[pl.ds(g * L, L)] = plsc.load_gather(inv_v, [d16])
      kv[pl.ds(g * L, L)] = s16

  def fire_rows(fs, fd, ra, rb, sa, sb):
    pltpu.async_copy(emb_hbm.at[fs], ra, sa)
    pltpu.async_copy(emb_hbm.at[fd], rb, sb)

  def wait_rows(fs, fd, ra, rb, sa, sb):
    pltpu.make_async_copy(emb_hbm.at[fs], ra, sa).wait()
    pltpu.make_async_copy(emb_hbm.at[fd], rb, sb).wait()

  ii17 = ii * 17

  def compute(rows_a, rows_b, kv, is_pos):
    def group_body(g):
      for j in range(L):
        e = g * L + j
        acc = jnp.zeros((L,), jnp.float32)
        for b in range(D // (2 * L)):
          a_bf = plsc.bitcast(rows_a[e, pl.ds(b * L, L)], jnp.bfloat16)
          b_bf = plsc.bitcast(rows_b[e, pl.ds(b * L, L)], jnp.bfloat16)
          p_lo, p_hi = plsc.unpack(a_bf * b_bf,
                                   format=plsc.PackFormat.INTERLEAVED)
          acc = acc + p_lo + p_hi
        plsc.store_scatter(tr_v, [ii + j * 17], acc)
      dots = plsc.load_gather(tr_v, [ii17])
      for k in range(1, L):
        dots = dots + plsc.load_gather(tr_v, [ii17 + k])
      keys = kv[pl.ds(g * L, L)]
      if is_pos:
        _seg_sum_update(keys, dots, ii, psum_v, pcnt_v, ktmp_v, vtmp_v,
                        ctmp_v)
      else:
        _seg_max_update(keys, dots, ii, nmax_v, ncnt_v, ktmp_v, vtmp_v)
      return 0

    lax.fori_loop(0, GROUPS, lambda g, c: group_body(g), 0)

  fire_edges(pos_hbm, 0, edges_p, sem_ep)

  def iter_body(i, carry):
    wait_edges(pos_hbm, edges_p, sem_ep)
    translate(edges_p, keys_p, fsrc_p, fdst_p)
    fire_rows(fsrc_p, fdst_p, rows_pa, rows_pb, sem_rpa, sem_rpb)
    fire_edges(neg_hbm, i, edges_n, sem_en)

    @pl.when(i > 0)
    def _():
      wait_rows(fsrc_n, fdst_n, rows_na, rows_nb, sem_rna, sem_rnb)
      compute(rows_na, rows_nb, keys_n, False)

    wait_edges(neg_hbm, edges_n, sem_en)
    translate(edges_n, keys_n, fsrc_n, fdst_n)
    fire_rows(fsrc_n, fdst_n, rows_na, rows_nb, sem_rna, sem_rnb)
    fire_edges(pos_hbm, jnp.minimum(i + 1, NCHUNK - 1),
               edges_p, sem_ep)

    wait_rows(fsrc_p, fdst_p, rows_pa, rows_pb, sem_rpa, sem_rpb)
    compute(rows_pa, rows_pb, keys_p, True)
    return carry

  lax.fori_loop(0, NCHUNK, iter_body, 0)

  wait_rows(fsrc_n, fdst_n, rows_na, rows_nb, sem_rna, sem_rnb)
  compute(rows_na, rows_nb, keys_n, False)
  wait_edges(pos_hbm, edges_p, sem_ep)

  pltpu.sync_copy(psum_v, psum_out.at[wid])
  pltpu.sync_copy(pcnt_v, pcnt_out.at[wid])
  pltpu.sync_copy(nmax_v, nmax_out.at[wid])
  pltpu.sync_copy(ncnt_v, ncnt_out.at[wid])


def _combine_body(psum_ref, pcnt_ref, nmax_ref, ncnt_ref, out_ref):
  p_sum = jnp.sum(psum_ref[...], axis=0, keepdims=True)
  p_cnt = jnp.sum(pcnt_ref[...], axis=0, keepdims=True)
  n_max = jnp.max(nmax_ref[...], axis=0, keepdims=True)
  n_cnt = jnp.sum(ncnt_ref[...], axis=0, keepdims=True)
  p_d = p_sum / jnp.maximum(p_cnt, 1.0)
  n_d = jnp.where(n_cnt > 0.0, n_max, 0.0)
  hinge = jnp.maximum(n_d - p_d + DELTA, 0.0)
  idx = lax.broadcasted_iota(jnp.int32, (1, SEG), 1)
  hinge = jnp.where(idx < N_NODES, hinge, 0.0)
  out_ref[...] = (jnp.sum(hinge) / N_NODES).reshape(1, 1)


def kernel(emb, inv_idx, pos_edges, neg_edges):
  emb_pk = lax.bitcast_convert_type(
      emb.astype(jnp.bfloat16).reshape(N_NODES, D // 2, 2), jnp.int32)
  psum, pcnt, nmax, ncnt = _sc_partials(
      emb_pk,
      inv_idx.astype(jnp.int32),
      pos_edges.astype(jnp.int32),
      neg_edges.astype(jnp.int32),
  )
  loss = pl.pallas_call(
      _combine_body,
      out_shape=jax.ShapeDtypeStruct((1, 1), jnp.float32),
  )(psum, pcnt, nmax, ncnt)
  return loss[0, 0]


# speedup vs baseline: 1.5915x; 1.5915x over previous
import functools

import jax
import jax.numpy as jnp
from jax import lax
from jax.experimental import pallas as pl
from jax.experimental.pallas import tpu as pltpu
from jax.experimental.pallas import tpu_sc as plsc

N_NODES = 10000
D = 128
E = 320000
DELTA = 1.0

SEG = 10240
NC = 2
NS = 16
L = 16
NW = NC * NS
EPW = E // NW
C = 80
NCHUNK = EPW // C
GROUPS = C // L
NEG_INIT = -3.0e38


def _shift(tmp_v, x, idx):
  tmp_v[pl.ds(0, L)] = x
  return plsc.load_gather(tmp_v, [idx])


def _seg_sum_update(keys, dots, ii, sum_v, cnt_v, ktmp_v, vtmp_v, ctmp_v):
  ks, vs = plsc.sort_key_val(keys, dots)
  ktmp_v[pl.ds(0, L)] = ks
  k_prev = plsc.load_gather(ktmp_v, [jnp.maximum(ii - 1, 0)])
  is_head = (ii == 0) | (ks != k_prev)
  head = plsc.cummax(jnp.where(is_head, ii, 0))
  csum = plsc.cumsum(vs)
  vtmp_v[pl.ds(0, L)] = csum
  s_prev = plsc.load_gather(vtmp_v, [jnp.maximum(head - 1, 0)])
  run_sum = csum - jnp.where(head > 0, s_prev, 0.0)
  run_cnt = (ii - head + 1).astype(jnp.float32)
  k_next = plsc.load_gather(ktmp_v, [jnp.minimum(ii + 1, L - 1)])
  is_tail = (ii == L - 1) | (ks != k_next)
  old_s = plsc.load_gather(sum_v, [ks])
  plsc.store_scatter(sum_v, [ks], old_s + run_sum, mask=is_tail)
  old_c = plsc.load_gather(cnt_v, [ks])
  plsc.store_scatter(cnt_v, [ks], old_c + run_cnt, mask=is_tail)


def _seg_max_update(keys, dots, ii, max_v, flag_v, ktmp_v, vtmp_v):
  vs1, ks1 = plsc.sort_key_val(dots, keys, descending=True)
  ks, vs = plsc.sort_key_val(ks1, vs1)
  ktmp_v[pl.ds(0, L)] = ks
  k_prev = plsc.load_gather(ktmp_v, [jnp.maximum(ii - 1, 0)])
  is_head = (ii == 0) | (ks != k_prev)
  old_m = plsc.load_gather(max_v, [ks])
  plsc.store_scatter(max_v, [ks], jnp.maximum(old_m, vs), mask=is_head)
  plsc.store_scatter(flag_v, [ks], jnp.ones((L,), jnp.float32), mask=is_head)


_MESH = plsc.VectorSubcoreMesh(core_axis_name="c", subcore_axis_name="s")


@functools.partial(
    pl.kernel,
    out_type=[jax.ShapeDtypeStruct((NW, SEG), jnp.float32)] * 4,
    mesh=_MESH,
    compiler_params=pltpu.CompilerParams(
        needs_layout_passes=False, use_tc_tiling_on_sc=False),
    scratch_types=[
        pltpu.VMEM((N_NODES,), jnp.int32),
        pltpu.VMEM((SEG,), jnp.float32),
        pltpu.VMEM((SEG,), jnp.float32),
        pltpu.VMEM((SEG,), jnp.float32),
        pltpu.VMEM((SEG,), jnp.float32),
        pltpu.VMEM((2, C), jnp.int32),
        pltpu.VMEM((2, C), jnp.int32),
        pltpu.VMEM((C,), jnp.int32),
        pltpu.VMEM((C,), jnp.int32),
        pltpu.VMEM((C,), jnp.int32),
        pltpu.VMEM((C,), jnp.int32),
        pltpu.VMEM((C,), jnp.int32),
        pltpu.VMEM((C,), jnp.int32),
        pltpu.VMEM((C, D // 2), jnp.int32),
        pltpu.VMEM((C, D // 2), jnp.int32),
        pltpu.VMEM((C, D // 2), jnp.int32),
        pltpu.VMEM((C, D // 2), jnp.int32),
        pltpu.VMEM((L,), jnp.int32),
        pltpu.VMEM((L,), jnp.float32),
        pltpu.VMEM((L,), jnp.float32),
        pltpu.SemaphoreType.DMA,
        pltpu.SemaphoreType.DMA,
        pltpu.SemaphoreType.DMA,
        pltpu.SemaphoreType.DMA,
        pltpu.SemaphoreType.DMA,
        pltpu.SemaphoreType.DMA,
    ],
)
def _sc_partials(emb_hbm, inv_hbm, pos_hbm, neg_hbm,
                 psum_out, pcnt_out, nmax_out, ncnt_out,
                 inv_v, psum_v, pcnt_v, nmax_v, ncnt_v,
                 edges_p, edges_n, keys_p, keys_n,
                 fsrc_p, fdst_p, fsrc_n, fdst_n,
                 rows_pa, rows_pb, rows_na, rows_nb,
                 ktmp_v, vtmp_v, ctmp_v,
                 sem_ep, sem_en, sem_rpa, sem_rpb, sem_rna, sem_rnb):
  wid = lax.axis_index("s") * NC + lax.axis_index("c")
  ii = lax.iota(jnp.int32, L)
  ebase = wid * EPW

  zero16 = jnp.zeros((L,), jnp.float32)
  neg16 = jnp.full((L,), NEG_INIT, jnp.float32)

  def init_body(i, carry):
    off = i * L
    psum_v[pl.ds(off, L)] = zero16
    pcnt_v[pl.ds(off, L)] = zero16
    nmax_v[pl.ds(off, L)] = neg16
    ncnt_v[pl.ds(off, L)] = zero16
    return carry

  lax.fori_loop(0, SEG // L, init_body, 0)

  pltpu.sync_copy(inv_hbm, inv_v)

  def fire_edges(e_hbm, ci, ebuf, sem):
    base = ebase + ci * C
    pltpu.async_copy(e_hbm.at[pl.ds(0, 2), pl.ds(base, C)], ebuf, sem)

  def wait_edges(e_hbm, ebuf, sem):
    pltpu.make_async_copy(e_hbm.at[pl.ds(0, 2), pl.ds(0, C)], ebuf,
                          sem).wait()

  def translate(ebuf, kv, fs, fd):
    for g in range(GROUPS):
      s16 = ebuf[0, pl.ds(g * L, L)]
      d16 = ebuf[1, pl.ds(g * L, L)]
      fs[pl.ds(g * L, L)] = plsc.load_gather(inv_v, [s16])
      fd[pl.ds(g * L, L)] = plsc.load_gather(inv_v, [d16])
      kv[pl.ds(g * L, L)] = s16

  def fire_rows(fs, fd, ra, rb, sa, sb):
    pltpu.async_copy(emb_hbm.at[fs], ra, sa)
    pltpu.async_copy(emb_hbm.at[fd], rb, sb)

  def wait_rows(fs, fd, ra, rb, sa, sb):
    pltpu.make_async_copy(emb_hbm.at[fs], ra, sa).wait()
    pltpu.make_async_copy(emb_hbm.at[fd], rb, sb).wait()

  def compute(rows_a, rows_b, kv, is_pos):
    def group_body(g):
      dots = jnp.zeros((L,), jnp.float32)
      for j in range(L):
        e = g * L + j
        acc = jnp.zeros((L,), jnp.float32)
        for b in range(D // (2 * L)):
          a_bf = plsc.bitcast(rows_a[e, pl.ds(b * L, L)], jnp.bfloat16)
          b_bf = plsc.bitcast(rows_b[e, pl.ds(b * L, L)], jnp.bfloat16)
          p_lo, p_hi = plsc.unpack(a_bf * b_bf,
                                   format=plsc.PackFormat.INTERLEAVED)
          acc = acc + p_lo + p_hi
        s = jnp.sum(acc)
        dots = jnp.where(ii == j, s, dots)
      keys = kv[pl.ds(g * L, L)]
      if is_pos:
        _seg_sum_update(keys, dots, ii, psum_v, pcnt_v, ktmp_v, vtmp_v,
                        ctmp_v)
      else:
        _seg_max_update(keys, dots, ii, nmax_v, ncnt_v, ktmp_v, vtmp_v)
      return 0

    lax.fori_loop(0, GROUPS, lambda g, c: group_body(g), 0)

  fire_edges(pos_hbm, 0, edges_p, sem_ep)

  def iter_body(i, carry):
    wait_edges(pos_hbm, edges_p, sem_ep)
    translate(edges_p, keys_p, fsrc_p, fdst_p)
    fire_rows(fsrc_p, fdst_p, rows_pa, rows_pb, sem_rpa, sem_rpb)
    fire_edges(neg_hbm, i, edges_n, sem_en)

    @pl.when(i > 0)
    def _():
      wait_rows(fsrc_n, fdst_n, rows_na, rows_nb, sem_rna, sem_rnb)
      compute(rows_na, rows_nb, keys_n, False)

    wait_edges(neg_hbm, edges_n, sem_en)
    translate(edges_n, keys_n, fsrc_n, fdst_n)
    fire_rows(fsrc_n, fdst_n, rows_na, rows_nb, sem_rna, sem_rnb)
    fire_edges(pos_hbm, jnp.minimum(i + 1, NCHUNK - 1),
               edges_p, sem_ep)

    wait_rows(fsrc_p, fdst_p, rows_pa, rows_pb, sem_rpa, sem_rpb)
    compute(rows_pa, rows_pb, keys_p, True)
    return carry

  lax.fori_loop(0, NCHUNK, iter_body, 0)

  wait_rows(fsrc_n, fdst_n, rows_na, rows_nb, sem_rna, sem_rnb)
  compute(rows_na, rows_nb, keys_n, False)
  wait_edges(pos_hbm, edges_p, sem_ep)

  pltpu.sync_copy(psum_v, psum_out.at[wid])
  pltpu.sync_copy(pcnt_v, pcnt_out.at[wid])
  pltpu.sync_copy(nmax_v, nmax_out.at[wid])
  pltpu.sync_copy(ncnt_v, ncnt_out.at[wid])


def _combine_body(psum_ref, pcnt_ref, nmax_ref, ncnt_ref, out_ref):
  p_sum = jnp.sum(psum_ref[...], axis=0, keepdims=True)
  p_cnt = jnp.sum(pcnt_ref[...], axis=0, keepdims=True)
  n_max = jnp.max(nmax_ref[...], axis=0, keepdims=True)
  n_cnt = jnp.sum(ncnt_ref[...], axis=0, keepdims=True)
  p_d = p_sum / jnp.maximum(p_cnt, 1.0)
  n_d = jnp.where(n_cnt > 0.0, n_max, 0.0)
  hinge = jnp.maximum(n_d - p_d + DELTA, 0.0)
  idx = lax.broadcasted_iota(jnp.int32, (1, SEG), 1)
  hinge = jnp.where(idx < N_NODES, hinge, 0.0)
  out_ref[...] = (jnp.sum(hinge) / N_NODES).reshape(1, 1)


def kernel(emb, inv_idx, pos_edges, neg_edges):
  emb_pk = lax.bitcast_convert_type(
      emb.astype(jnp.bfloat16).reshape(N_NODES, D // 2, 2), jnp.int32)
  psum, pcnt, nmax, ncnt = _sc_partials(
      emb_pk,
      inv_idx.astype(jnp.int32),
      pos_edges.astype(jnp.int32),
      neg_edges.astype(jnp.int32),
  )
  loss = pl.pallas_call(
      _combine_body,
      out_shape=jax.ShapeDtypeStruct((1, 1), jnp.float32),
  )(psum, pcnt, nmax, ncnt)
  return loss[0, 0]
